# baseline (speedup 1.0000x reference)
import functools

import jax
import jax.numpy as jnp
from jax import lax
from jax.experimental import pallas as pl
from jax.experimental.pallas import tpu as pltpu
from jax.experimental.pallas import tpu_sc as plsc

_N1 = 6
_N2 = 3
_D = 128
_ROWS = 200
_NOUT = _N1 * _N2
_NPAD = 32


def _body(idx1_hbm, idx2_hbm, xflat_hbm, out_hbm, idx1_v, idx2_v, flat_v,
          rows_v, sem):
    c = lax.axis_index("c")
    s = lax.axis_index("s")

    @pl.when((c == 0) & (s == 0))
    def _():
        pltpu.sync_copy(idx1_hbm, idx1_v.at[pl.ds(0, _N1)])
        pltpu.sync_copy(idx2_hbm, idx2_v.at[pl.ds(0, _N2)])
        v1 = idx1_v[...]
        v2 = idx2_v[...]
        three = lax.full((16,), _N2, jnp.int32)
        rows = lax.full((16,), _ROWS, jnp.int32)
        k0 = lax.broadcasted_iota(jnp.int32, (16,), 0)
        i0 = lax.div(k0, three)
        j0 = lax.sub(k0, lax.mul(i0, three))
        flat0 = lax.add(
            lax.mul(jnp.take_along_axis(v1, i0, axis=0), rows),
            jnp.take_along_axis(v2, j0, axis=0))
        flat_v[pl.ds(0, 16)] = flat0
        k1 = lax.add(k0, lax.full((16,), 16, jnp.int32))
        q1 = lax.div(k1, three)
        i1 = lax.min(q1, lax.full((16,), _N1 - 1, jnp.int32))
        j1 = lax.sub(k1, lax.mul(q1, three))
        flat1 = lax.add(
            lax.mul(jnp.take_along_axis(v1, i1, axis=0), rows),
            jnp.take_along_axis(v2, j1, axis=0))
        flat_v[pl.ds(16, 16)] = flat1
        pltpu.async_copy(xflat_hbm.at[flat_v], rows_v, sem).wait()
        pltpu.sync_copy(rows_v, out_hbm)


_sc_gather = functools.partial(
    pl.kernel,
    mesh=plsc.VectorSubcoreMesh(core_axis_name="c", subcore_axis_name="s"),
    out_type=jax.ShapeDtypeStruct((_NPAD, _D), jnp.float32),
    scratch_types=[
        pltpu.VMEM((16,), jnp.int32),
        pltpu.VMEM((16,), jnp.int32),
        pltpu.VMEM((_NPAD,), jnp.int32),
        pltpu.VMEM((_NPAD, _D), jnp.float32),
        pltpu.SemaphoreType.DMA,
    ],
)(_body)


@jax.jit
def kernel(x, index1, index2):
    xflat = x.reshape(-1, _D)
    y = _sc_gather(index1.reshape(_N1), index2, xflat)
    return y[:_NOUT].reshape(_N1, _N2, _D)


# speedup vs baseline: 1.1008x; 1.1008x over previous
import functools

import jax
import jax.numpy as jnp
from jax import lax
from jax.experimental import pallas as pl
from jax.experimental.pallas import tpu as pltpu
from jax.experimental.pallas import tpu_sc as plsc

_N1 = 6
_N2 = 3
_D = 128
_ROWS = 200
_NOUT = _N1 * _N2


def _body(idx1_hbm, idx2_hbm, xflat_hbm, out_hbm, idx1_v, idx2_v, rows_v,
          sem1, sem2):
    cp1 = pltpu.make_async_copy(idx1_hbm, idx1_v.at[pl.ds(0, _N1)], sem1)
    cp2 = pltpu.make_async_copy(idx2_hbm, idx2_v.at[pl.ds(0, _N2)], sem2)
    cp1.start()
    cp2.start()
    cp1.wait()
    cp2.wait()
    v1 = idx1_v[...]
    v2 = idx2_v[...]
    three = lax.full((16,), _N2, jnp.int32)
    rows = lax.full((16,), _ROWS, jnp.int32)
    k0 = lax.broadcasted_iota(jnp.int32, (16,), 0)
    i0 = lax.div(k0, three)
    j0 = lax.sub(k0, lax.mul(i0, three))
    flat0 = lax.add(
        lax.mul(jnp.take_along_axis(v1, i0, axis=0), rows),
        jnp.take_along_axis(v2, j0, axis=0))
    k1 = lax.add(k0, lax.full((16,), 16, jnp.int32))
    q1 = lax.div(k1, three)
    i1 = lax.min(q1, lax.full((16,), _N1 - 1, jnp.int32))
    j1 = lax.sub(k1, lax.mul(q1, three))
    flat1 = lax.add(
        lax.mul(jnp.take_along_axis(v1, i1, axis=0), rows),
        jnp.take_along_axis(v2, j1, axis=0))
    g0 = pltpu.make_async_copy(
        xflat_hbm.at[flat0], rows_v.at[pl.ds(0, 16)], sem1)
    g1 = pltpu.make_async_copy(
        xflat_hbm.at[flat1], rows_v.at[pl.ds(16, 16)], sem2)
    g0.start()
    g1.start()
    g0.wait()
    g1.wait()
    pltpu.sync_copy(rows_v.at[pl.ds(0, _NOUT)], out_hbm)


_sc_gather = functools.partial(
    pl.kernel,
    mesh=plsc.VectorSubcoreMesh(
        core_axis_name="c", subcore_axis_name="s", num_cores=1,
        num_subcores=1),
    out_type=jax.ShapeDtypeStruct((_NOUT, _D), jnp.float32),
    scratch_types=[
        pltpu.VMEM((16,), jnp.int32),
        pltpu.VMEM((16,), jnp.int32),
        pltpu.VMEM((32, _D), jnp.float32),
        pltpu.SemaphoreType.DMA,
        pltpu.SemaphoreType.DMA,
    ],
)(_body)


@jax.jit
def kernel(x, index1, index2):
    xflat = x.reshape(-1, _D)
    y = _sc_gather(index1.reshape(_N1), index2, xflat)
    return y.reshape(_N1, _N2, _D)


# speedup vs baseline: 1.1801x; 1.0720x over previous
import functools

import jax
import jax.numpy as jnp
from jax.experimental import pallas as pl
from jax.experimental.pallas import tpu as pltpu
from jax.experimental.pallas import tpu_sc as plsc

_N1 = 6
_N2 = 3
_D = 128
_ROWS = 200
_NOUT = _N1 * _N2


def _body(idx1_hbm, idx2_hbm, xflat_hbm, out_hbm, idx1_s, idx2_s, sem1, sem2,
          gsem):
    cp1 = pltpu.make_async_copy(idx1_hbm, idx1_s, sem1)
    cp2 = pltpu.make_async_copy(idx2_hbm, idx2_s, sem2)
    cp1.start()
    cp2.start()
    cp1.wait()
    cp2.wait()
    copies = []
    for k in range(_NOUT):
        flat = idx1_s[k // _N2] * _ROWS + idx2_s[k % _N2]
        cp = pltpu.make_async_copy(
            xflat_hbm.at[pl.ds(flat, 1)], out_hbm.at[pl.ds(k, 1)], gsem)
        cp.start()
        copies.append(cp)
    for cp in copies:
        cp.wait()


_sc_gather = functools.partial(
    pl.kernel,
    mesh=plsc.ScalarSubcoreMesh(axis_name="c", num_cores=1),
    out_type=jax.ShapeDtypeStruct((_NOUT, _D), jnp.float32),
    scratch_types=[
        pltpu.SMEM((_N1,), jnp.int32),
        pltpu.SMEM((_N2,), jnp.int32),
        pltpu.SemaphoreType.DMA,
        pltpu.SemaphoreType.DMA,
        pltpu.SemaphoreType.DMA,
    ],
)(_body)


@jax.jit
def kernel(x, index1, index2):
    xflat = x.reshape(-1, _D)
    y = _sc_gather(index1.reshape(_N1), index2, xflat)
    return y.reshape(_N1, _N2, _D)


# speedup vs baseline: 1.1827x; 1.0023x over previous
import functools

import jax
import jax.numpy as jnp
from jax import lax
from jax.experimental import pallas as pl
from jax.experimental.pallas import tpu as pltpu
from jax.experimental.pallas import tpu_sc as plsc

_N1 = 6
_N2 = 3
_D = 128
_ROWS = 200
_NOUT = _N1 * _N2


def _body(idx1_hbm, idx2_hbm, xflat_hbm, out_hbm, idx1_s, idx2_s, sem1, sem2,
          gsem):
    cp1 = pltpu.make_async_copy(idx1_hbm, idx1_s, sem1)
    cp2 = pltpu.make_async_copy(idx2_hbm, idx2_s, sem2)
    cp1.start()
    cp2.start()
    cp1.wait()
    cp2.wait()

    def issue(k, carry):
        i = k // _N2
        j = k - i * _N2
        flat = idx1_s[i] * _ROWS + idx2_s[j]
        pltpu.make_async_copy(
            xflat_hbm.at[pl.ds(flat, 1)], out_hbm.at[pl.ds(k, 1)], gsem
        ).start()
        return carry

    def drain(k, carry):
        pltpu.make_async_copy(
            xflat_hbm.at[pl.ds(0, 1)], out_hbm.at[pl.ds(0, 1)], gsem
        ).wait()
        return carry

    lax.fori_loop(0, _NOUT, issue, 0, unroll=False)
    lax.fori_loop(0, _NOUT, drain, 0, unroll=False)


_sc_gather = functools.partial(
    pl.kernel,
    mesh=plsc.ScalarSubcoreMesh(axis_name="c", num_cores=1),
    out_type=jax.ShapeDtypeStruct((_NOUT, _D), jnp.float32),
    scratch_types=[
        pltpu.SMEM((_N1,), jnp.int32),
        pltpu.SMEM((_N2,), jnp.int32),
        pltpu.SemaphoreType.DMA,
        pltpu.SemaphoreType.DMA,
        pltpu.SemaphoreType.DMA,
    ],
)(_body)


@jax.jit
def kernel(x, index1, index2):
    xflat = x.reshape(-1, _D)
    y = _sc_gather(index1.reshape(_N1), index2, xflat)
    return y.reshape(_N1, _N2, _D)


# speedup vs baseline: 6.6594x; 5.6304x over previous
import functools

import jax
import jax.numpy as jnp
from jax.experimental import pallas as pl
from jax.experimental.pallas import tpu as pltpu

_N1 = 6
_N2 = 3
_D = 128
_ROWS = 200
_NOUT = _N1 * _N2


def _body(idx1_s, idx2_s, x_hbm, out_hbm, sem):
    copies = []
    for k in range(_NOUT):
        flat = idx1_s[k // _N2] * _ROWS + idx2_s[k % _N2]
        cp = pltpu.make_async_copy(
            x_hbm.at[pl.ds(flat, 1)], out_hbm.at[pl.ds(k, 1)], sem)
        cp.start()
        copies.append(cp)
    for cp in copies:
        cp.wait()


_tc_gather = pl.pallas_call(
    _body,
    grid_spec=pltpu.PrefetchScalarGridSpec(
        num_scalar_prefetch=2,
        grid=(1,),
        in_specs=[pl.BlockSpec(memory_space=pl.ANY)],
        out_specs=pl.BlockSpec(memory_space=pl.ANY),
        scratch_shapes=[pltpu.SemaphoreType.DMA],
    ),
    out_shape=jax.ShapeDtypeStruct((_NOUT, _D), jnp.float32),
)


@jax.jit
def kernel(x, index1, index2):
    xflat = x.reshape(-1, _D)
    y = _tc_gather(index1.reshape(_N1), index2, xflat)
    return y.reshape(_N1, _N2, _D)
